# Initial kernel scaffold; baseline (speedup 1.0000x reference)
import jax
import jax.numpy as jnp
from jax.experimental import pallas as pl


def kernel(x, edge_index, pyg_batch, c1_W1, c1_b1, c1_g, c1_be, c1_W2, c1_b2, c2_W1, c2_b1, c2_g, c2_be, c2_W2, c2_b2, c3_W1, c3_b1, c3_g, c3_be, c3_W2, c3_b2, lin1_W, lin1_b, lin2_W, lin2_b):
    raise NotImplementedError("write your pallas kernel here")


# speedup vs baseline: 5.3192x; 5.3192x over previous
import functools

import jax
import jax.numpy as jnp
from jax import lax
from jax.experimental import pallas as pl
from jax.experimental.pallas import tpu as pltpu
from jax.experimental.pallas import tpu_sc as plsc

N = 10000
E = 320000
D = 128
B = 128

NC = 2
NS = 16
NW = NC * NS

CHUNK = 128
E_PAD = 323584
CH_PER_TILE = E_PAD // (NW * CHUNK)
ACC_R = 10240
ZB = 64

BN_SCALE = 1.0 / (1.0 + 1e-5) ** 0.5

BLK = 1000
GRID = N // BLK


def _segsum_body(h_hbm, src_hbm, dst_hbm, out_hbm, acc_sh, zbuf, sidx, didx, rows, sem):
    c = lax.axis_index("c")
    s = lax.axis_index("s")
    wid = s * NC + c

    def _zr(i, carry):
        r = i // 8
        j = i % 8
        zbuf[r, pl.ds(j * 16, 16)] = jnp.zeros((16,), jnp.float32)
        return carry

    lax.fori_loop(0, ZB * 8, _zr, 0)

    zbase = s * (ZB * 10)

    def _zcp(i, carry):
        pltpu.sync_copy(zbuf, acc_sh.at[pl.ds(zbase + i * ZB, ZB), :])
        return carry

    lax.fori_loop(0, 10, _zcp, 0)
    plsc.subcore_barrier()

    ch0 = wid * CH_PER_TILE

    def _edge(i, carry):
        ch = ch0 + i
        pltpu.sync_copy(src_hbm.at[pl.ds(ch * CHUNK, CHUNK)], sidx)
        pltpu.async_copy(h_hbm.at[sidx], rows, sem).wait()
        pltpu.sync_copy(dst_hbm.at[pl.ds(ch, 1), :], didx)
        pltpu.sync_copy(rows, acc_sh.at[didx.at[0]], add=True)
        return carry

    lax.fori_loop(0, CH_PER_TILE, _edge, 0)
    plsc.subcore_barrier()

    r0 = s * (ACC_R // NS)
    pltpu.sync_copy(acc_sh.at[pl.ds(r0, ACC_R // NS), :],
                    out_hbm.at[c, pl.ds(r0, ACC_R // NS), :])


_segsum = pl.kernel(
    _segsum_body,
    out_type=jax.ShapeDtypeStruct((NC, ACC_R, D), jnp.float32),
    mesh=plsc.VectorSubcoreMesh(core_axis_name="c", subcore_axis_name="s",
                                num_cores=NC, num_subcores=NS),
    scratch_types=[
        pltpu.VMEM_SHARED((ACC_R, D), jnp.float32),
        pltpu.VMEM((ZB, D), jnp.float32),
        pltpu.VMEM((CHUNK,), jnp.int32),
        pltpu.VMEM((1, CHUNK), jnp.int32),
        pltpu.VMEM((CHUNK, D), jnp.float32),
        pltpu.SemaphoreType.DMA,
    ],
)


def _mlp_pool_body(xb, a0b, a1b, idsb, W1, b1, gb, beb, W2, b2, h_out, p_out):
    i = pl.program_id(0)
    y = xb[...] + a0b[...] + a1b[...]
    h = jnp.dot(y, W1[...], preferred_element_type=jnp.float32) + b1[...]
    h = h * (gb[...] * BN_SCALE) + beb[...]
    h = jnp.maximum(h, 0.0)
    h = jnp.dot(h, W2[...], preferred_element_type=jnp.float32) + b2[...]
    h = jnp.maximum(h, 0.0)
    h_out[...] = h
    ids = idsb[0, 0, :]
    onehot = (ids[None, :] == lax.broadcasted_iota(jnp.int32, (B, BLK), 0)
              ).astype(jnp.float32)
    pc = jnp.dot(onehot, h, preferred_element_type=jnp.float32)

    @pl.when(i == 0)
    def _():
        p_out[...] = pc

    @pl.when(i != 0)
    def _():
        p_out[...] += pc


_mlp_pool = pl.pallas_call(
    _mlp_pool_body,
    grid=(GRID,),
    in_specs=[
        pl.BlockSpec((BLK, D), lambda i: (i, 0)),
        pl.BlockSpec((BLK, D), lambda i: (i, 0)),
        pl.BlockSpec((BLK, D), lambda i: (i, 0)),
        pl.BlockSpec((1, 1, BLK), lambda i: (i, 0, 0)),
        pl.BlockSpec((D, D), lambda i: (0, 0)),
        pl.BlockSpec((1, D), lambda i: (0, 0)),
        pl.BlockSpec((1, D), lambda i: (0, 0)),
        pl.BlockSpec((1, D), lambda i: (0, 0)),
        pl.BlockSpec((D, D), lambda i: (0, 0)),
        pl.BlockSpec((1, D), lambda i: (0, 0)),
    ],
    out_specs=[
        pl.BlockSpec((BLK, D), lambda i: (i, 0)),
        pl.BlockSpec((B, D), lambda i: (0, 0)),
    ],
    out_shape=[
        jax.ShapeDtypeStruct((N, D), jnp.float32),
        jax.ShapeDtypeStruct((B, D), jnp.float32),
    ],
)


def _head_body(p1, p2, p3, W1, b1, w2r, b2, out):
    h = jnp.concatenate((p1[...], p2[...], p3[...]), axis=1)
    h = jnp.dot(h, W1[...], preferred_element_type=jnp.float32) + b1[...]
    h = jnp.where(h > 0.0, h, 0.01 * h)
    o = lax.dot_general(w2r[...], h, (((1,), (1,)), ((), ())),
                        preferred_element_type=jnp.float32)
    out[...] = o + b2[...]


_head = pl.pallas_call(
    _head_body,
    out_shape=jax.ShapeDtypeStruct((1, B), jnp.float32),
)


def kernel(x, edge_index, pyg_batch,
           c1_W1, c1_b1, c1_g, c1_be, c1_W2, c1_b2,
           c2_W1, c2_b1, c2_g, c2_be, c2_W2, c2_b2,
           c3_W1, c3_b1, c3_g, c3_be, c3_W2, c3_b2,
           lin1_W, lin1_b, lin2_W, lin2_b):
    pad = E_PAD - E
    pad_ar = jnp.arange(pad, dtype=jnp.int32)
    src_p = jnp.concatenate((edge_index[0], (pad_ar * 37) % N))
    dst_p = jnp.concatenate((edge_index[1], N + (pad_ar % 128)))
    dst2d = dst_p.reshape(E_PAD // CHUNK, CHUNK)
    ids3d = pyg_batch.reshape(GRID, 1, BLK)

    r = lambda v: v.reshape(1, -1)

    h = x
    pools = []
    for (W1, b1, g, be, W2, b2) in (
        (c1_W1, c1_b1, c1_g, c1_be, c1_W2, c1_b2),
        (c2_W1, c2_b1, c2_g, c2_be, c2_W2, c2_b2),
        (c3_W1, c3_b1, c3_g, c3_be, c3_W2, c3_b2),
    ):
        part = _segsum(h, src_p, dst2d)
        h, p = _mlp_pool(h, part[0], part[1], ids3d,
                         W1, r(b1), r(g), r(be), W2, r(b2))
        pools.append(p)

    out = _head(pools[0], pools[1], pools[2],
                lin1_W, r(lin1_b), lin2_W.reshape(1, -1), r(lin2_b))
    return out.reshape(B)


# speedup vs baseline: 9.1923x; 1.7281x over previous
import functools

import jax
import jax.numpy as jnp
from jax import lax
from jax.experimental import pallas as pl
from jax.experimental.pallas import tpu as pltpu
from jax.experimental.pallas import tpu_sc as plsc

N = 10000
E = 320000
D = 128
B = 128

NC = 2
NS = 16
NW = NC * NS

CHUNK = 64
E_PAD = 327680
CH_PER_TILE = E_PAD // (NW * CHUNK)
ACC_R = 10240

BN_SCALE = 1.0 / (1.0 + 1e-5) ** 0.5

BLK = 1000
GRID = N // BLK


def _segsum_body(h_hbm, src_hbm, dst_hbm, out_hbm, acc_sh,
                 sidx, didx, rows0, rows1, gsem0, gsem1, isem):
    c = lax.axis_index("c")
    s = lax.axis_index("s")
    wid = s * NC + c

    ch0 = wid * CH_PER_TILE
    i0 = pltpu.async_copy(src_hbm.at[pl.ds(ch0 * CHUNK, CH_PER_TILE * CHUNK)],
                          sidx, isem)
    i1 = pltpu.async_copy(dst_hbm.at[pl.ds(ch0, CH_PER_TILE), :], didx, isem)

    def _zr(i, carry):
        r = i // 8
        j = i % 8
        rows0[r, pl.ds(j * 16, 16)] = jnp.zeros((16,), jnp.float32)
        return carry

    lax.fori_loop(0, CHUNK * 8, _zr, 0)

    zbase = s * (ACC_R // NS)

    def _zcp(i, carry):
        pltpu.sync_copy(rows0, acc_sh.at[pl.ds(zbase + i * CHUNK, CHUNK), :])
        return carry

    lax.fori_loop(0, ACC_R // NS // CHUNK, _zcp, 0)
    i0.wait()
    i1.wait()
    plsc.subcore_barrier()

    def _gather(ch, rbuf, sem):
        pltpu.async_copy(h_hbm.at[sidx.at[pl.ds(ch * CHUNK, CHUNK)]], rbuf, sem)

    def _gwait(rbuf, sem):
        pltpu.make_async_copy(h_hbm.at[pl.ds(0, CHUNK), :], rbuf, sem).wait()

    def _scatter(ch, rbuf):
        pltpu.sync_copy(rbuf, acc_sh.at[didx.at[ch]], add=True)

    _gather(0, rows0, gsem0)

    def _pair(i, carry):
        nxt = 2 * i + 1
        _gather(nxt, rows1, gsem1)
        _gwait(rows0, gsem0)
        _scatter(2 * i, rows0)

        @pl.when(nxt + 1 < CH_PER_TILE)
        def _():
            _gather(nxt + 1, rows0, gsem0)

        _gwait(rows1, gsem1)
        _scatter(nxt, rows1)
        return carry

    lax.fori_loop(0, CH_PER_TILE // 2, _pair, 0)
    plsc.subcore_barrier()

    r0 = s * (ACC_R // NS)
    pltpu.sync_copy(acc_sh.at[pl.ds(r0, ACC_R // NS), :],
                    out_hbm.at[c, pl.ds(r0, ACC_R // NS), :])


_segsum = pl.kernel(
    _segsum_body,
    out_type=jax.ShapeDtypeStruct((NC, ACC_R, D), jnp.float32),
    mesh=plsc.VectorSubcoreMesh(core_axis_name="c", subcore_axis_name="s",
                                num_cores=NC, num_subcores=NS),
    scratch_types=[
        pltpu.VMEM_SHARED((ACC_R, D), jnp.float32),
        pltpu.VMEM((CH_PER_TILE * CHUNK,), jnp.int32),
        pltpu.VMEM((CH_PER_TILE, CHUNK), jnp.int32),
        pltpu.VMEM((CHUNK, D), jnp.float32),
        pltpu.VMEM((CHUNK, D), jnp.float32),
        pltpu.SemaphoreType.DMA,
        pltpu.SemaphoreType.DMA,
        pltpu.SemaphoreType.DMA,
    ],
)


def _mlp_pool_body(xb, a0b, a1b, idsb, W1, b1, gb, beb, W2, b2, h_out, p_out):
    i = pl.program_id(0)
    y = xb[...] + a0b[...] + a1b[...]
    h = jnp.dot(y, W1[...], preferred_element_type=jnp.float32) + b1[...]
    h = h * (gb[...] * BN_SCALE) + beb[...]
    h = jnp.maximum(h, 0.0)
    h = jnp.dot(h, W2[...], preferred_element_type=jnp.float32) + b2[...]
    h = jnp.maximum(h, 0.0)
    h_out[...] = h
    ids = idsb[0, 0, :]
    onehot = (ids[None, :] == lax.broadcasted_iota(jnp.int32, (B, BLK), 0)
              ).astype(jnp.float32)
    pc = jnp.dot(onehot, h, preferred_element_type=jnp.float32)

    @pl.when(i == 0)
    def _():
        p_out[...] = pc

    @pl.when(i != 0)
    def _():
        p_out[...] += pc


_mlp_pool = pl.pallas_call(
    _mlp_pool_body,
    grid=(GRID,),
    in_specs=[
        pl.BlockSpec((BLK, D), lambda i: (i, 0)),
        pl.BlockSpec((BLK, D), lambda i: (i, 0)),
        pl.BlockSpec((BLK, D), lambda i: (i, 0)),
        pl.BlockSpec((1, 1, BLK), lambda i: (i, 0, 0)),
        pl.BlockSpec((D, D), lambda i: (0, 0)),
        pl.BlockSpec((1, D), lambda i: (0, 0)),
        pl.BlockSpec((1, D), lambda i: (0, 0)),
        pl.BlockSpec((1, D), lambda i: (0, 0)),
        pl.BlockSpec((D, D), lambda i: (0, 0)),
        pl.BlockSpec((1, D), lambda i: (0, 0)),
    ],
    out_specs=[
        pl.BlockSpec((BLK, D), lambda i: (i, 0)),
        pl.BlockSpec((B, D), lambda i: (0, 0)),
    ],
    out_shape=[
        jax.ShapeDtypeStruct((N, D), jnp.float32),
        jax.ShapeDtypeStruct((B, D), jnp.float32),
    ],
)


def _head_body(p1, p2, p3, W1, b1, w2r, b2, out):
    h = jnp.concatenate((p1[...], p2[...], p3[...]), axis=1)
    h = jnp.dot(h, W1[...], preferred_element_type=jnp.float32) + b1[...]
    h = jnp.where(h > 0.0, h, 0.01 * h)
    o = lax.dot_general(w2r[...], h, (((1,), (1,)), ((), ())),
                        preferred_element_type=jnp.float32)
    out[...] = o + b2[...]


_head = pl.pallas_call(
    _head_body,
    out_shape=jax.ShapeDtypeStruct((1, B), jnp.float32),
)


def kernel(x, edge_index, pyg_batch,
           c1_W1, c1_b1, c1_g, c1_be, c1_W2, c1_b2,
           c2_W1, c2_b1, c2_g, c2_be, c2_W2, c2_b2,
           c3_W1, c3_b1, c3_g, c3_be, c3_W2, c3_b2,
           lin1_W, lin1_b, lin2_W, lin2_b):
    pad = E_PAD - E
    pad_ar = jnp.arange(pad, dtype=jnp.int32)
    src_p = jnp.concatenate((edge_index[0], (pad_ar * 37) % N))
    dst_p = jnp.concatenate((edge_index[1], N + (pad_ar % 128)))
    dst2d = dst_p.reshape(E_PAD // CHUNK, CHUNK)
    ids3d = pyg_batch.reshape(GRID, 1, BLK)

    r = lambda v: v.reshape(1, -1)

    h = x
    pools = []
    for (W1, b1, g, be, W2, b2) in (
        (c1_W1, c1_b1, c1_g, c1_be, c1_W2, c1_b2),
        (c2_W1, c2_b1, c2_g, c2_be, c2_W2, c2_b2),
        (c3_W1, c3_b1, c3_g, c3_be, c3_W2, c3_b2),
    ):
        part = _segsum(h, src_p, dst2d)
        h, p = _mlp_pool(h, part[0], part[1], ids3d,
                         W1, r(b1), r(g), r(be), W2, r(b2))
        pools.append(p)

    out = _head(pools[0], pools[1], pools[2],
                lin1_W, r(lin1_b), lin2_W.reshape(1, -1), r(lin2_b))
    return out.reshape(B)


# speedup vs baseline: 9.7891x; 1.0649x over previous
import functools

import jax
import jax.numpy as jnp
from jax import lax
from jax.experimental import pallas as pl
from jax.experimental.pallas import tpu as pltpu
from jax.experimental.pallas import tpu_sc as plsc

N = 10000
E = 320000
D = 128
B = 128

NC = 2
NS = 16
NW = NC * NS

CHUNK = 80
E_PAD = 327680
CH_PER_TILE = E_PAD // (NW * CHUNK)
ACC_R = 10240

BN_SCALE = 1.0 / (1.0 + 1e-5) ** 0.5

BLK = 1000
GRID = N // BLK


def _segsum_body(h_hbm, src_hbm, dst_hbm, out_hbm, acc_sh,
                 sidx, didx, rows0, rows1, gsem0, gsem1, isem):
    c = lax.axis_index("c")
    s = lax.axis_index("s")
    wid = s * NC + c

    ch0 = wid * CH_PER_TILE
    i0 = pltpu.async_copy(src_hbm.at[pl.ds(ch0 * CHUNK, CH_PER_TILE * CHUNK)],
                          sidx, isem)
    i1 = pltpu.async_copy(dst_hbm.at[pl.ds(ch0, CH_PER_TILE), :], didx, isem)

    def _zr(i, carry):
        r = i // 8
        j = i % 8
        rows0[r, pl.ds(j * 16, 16)] = jnp.zeros((16,), jnp.float32)
        return carry

    lax.fori_loop(0, CHUNK * 8, _zr, 0)

    zbase = s * (ACC_R // NS)

    def _zcp(i, carry):
        pltpu.sync_copy(rows0, acc_sh.at[pl.ds(zbase + i * CHUNK, CHUNK), :])
        return carry

    lax.fori_loop(0, ACC_R // NS // CHUNK, _zcp, 0)
    i0.wait()
    i1.wait()
    plsc.subcore_barrier()

    def _gather(ch, rbuf, sem):
        pltpu.async_copy(h_hbm.at[sidx.at[pl.ds(ch * CHUNK, CHUNK)]], rbuf, sem)

    def _gwait(rbuf, sem):
        pltpu.make_async_copy(h_hbm.at[pl.ds(0, CHUNK), :], rbuf, sem).wait()

    def _scatter(ch, rbuf):
        pltpu.sync_copy(rbuf, acc_sh.at[didx.at[ch]], add=True)

    _gather(0, rows0, gsem0)

    def _pair(i, carry):
        nxt = 2 * i + 1
        _gather(nxt, rows1, gsem1)
        _gwait(rows0, gsem0)
        _scatter(2 * i, rows0)

        @pl.when(nxt + 1 < CH_PER_TILE)
        def _():
            _gather(nxt + 1, rows0, gsem0)

        _gwait(rows1, gsem1)
        _scatter(nxt, rows1)
        return carry

    lax.fori_loop(0, CH_PER_TILE // 2, _pair, 0)
    plsc.subcore_barrier()

    r0 = s * (ACC_R // NS)
    pltpu.sync_copy(acc_sh.at[pl.ds(r0, ACC_R // NS), :],
                    out_hbm.at[c, pl.ds(r0, ACC_R // NS), :])


_segsum = pl.kernel(
    _segsum_body,
    out_type=jax.ShapeDtypeStruct((NC, ACC_R, D), jnp.float32),
    mesh=plsc.VectorSubcoreMesh(core_axis_name="c", subcore_axis_name="s",
                                num_cores=NC, num_subcores=NS),
    scratch_types=[
        pltpu.VMEM_SHARED((ACC_R, D), jnp.float32),
        pltpu.VMEM((CH_PER_TILE * CHUNK,), jnp.int32),
        pltpu.VMEM((CH_PER_TILE, CHUNK), jnp.int32),
        pltpu.VMEM((CHUNK, D), jnp.float32),
        pltpu.VMEM((CHUNK, D), jnp.float32),
        pltpu.SemaphoreType.DMA,
        pltpu.SemaphoreType.DMA,
        pltpu.SemaphoreType.DMA,
    ],
)


def _mlp_pool_body(xb, a0b, a1b, idsb, W1, b1, gb, beb, W2, b2, h_out, p_out):
    i = pl.program_id(0)
    y = xb[...] + a0b[...] + a1b[...]
    h = jnp.dot(y, W1[...], preferred_element_type=jnp.float32) + b1[...]
    h = h * (gb[...] * BN_SCALE) + beb[...]
    h = jnp.maximum(h, 0.0)
    h = jnp.dot(h, W2[...], preferred_element_type=jnp.float32) + b2[...]
    h = jnp.maximum(h, 0.0)
    h_out[...] = h
    ids = idsb[0, 0, :]
    onehot = (ids[None, :] == lax.broadcasted_iota(jnp.int32, (B, BLK), 0)
              ).astype(jnp.float32)
    pc = jnp.dot(onehot, h, preferred_element_type=jnp.float32)

    @pl.when(i == 0)
    def _():
        p_out[...] = pc

    @pl.when(i != 0)
    def _():
        p_out[...] += pc


_mlp_pool = pl.pallas_call(
    _mlp_pool_body,
    grid=(GRID,),
    in_specs=[
        pl.BlockSpec((BLK, D), lambda i: (i, 0)),
        pl.BlockSpec((BLK, D), lambda i: (i, 0)),
        pl.BlockSpec((BLK, D), lambda i: (i, 0)),
        pl.BlockSpec((1, 1, BLK), lambda i: (i, 0, 0)),
        pl.BlockSpec((D, D), lambda i: (0, 0)),
        pl.BlockSpec((1, D), lambda i: (0, 0)),
        pl.BlockSpec((1, D), lambda i: (0, 0)),
        pl.BlockSpec((1, D), lambda i: (0, 0)),
        pl.BlockSpec((D, D), lambda i: (0, 0)),
        pl.BlockSpec((1, D), lambda i: (0, 0)),
    ],
    out_specs=[
        pl.BlockSpec((BLK, D), lambda i: (i, 0)),
        pl.BlockSpec((B, D), lambda i: (0, 0)),
    ],
    out_shape=[
        jax.ShapeDtypeStruct((N, D), jnp.float32),
        jax.ShapeDtypeStruct((B, D), jnp.float32),
    ],
)


def _head_body(p1, p2, p3, W1, b1, w2r, b2, out):
    h = jnp.concatenate((p1[...], p2[...], p3[...]), axis=1)
    h = jnp.dot(h, W1[...], preferred_element_type=jnp.float32) + b1[...]
    h = jnp.where(h > 0.0, h, 0.01 * h)
    o = lax.dot_general(w2r[...], h, (((1,), (1,)), ((), ())),
                        preferred_element_type=jnp.float32)
    out[...] = o + b2[...]


_head = pl.pallas_call(
    _head_body,
    out_shape=jax.ShapeDtypeStruct((1, B), jnp.float32),
)


def kernel(x, edge_index, pyg_batch,
           c1_W1, c1_b1, c1_g, c1_be, c1_W2, c1_b2,
           c2_W1, c2_b1, c2_g, c2_be, c2_W2, c2_b2,
           c3_W1, c3_b1, c3_g, c3_be, c3_W2, c3_b2,
           lin1_W, lin1_b, lin2_W, lin2_b):
    pad = E_PAD - E
    pad_ar = jnp.arange(pad, dtype=jnp.int32)
    src_p = jnp.concatenate((edge_index[0], (pad_ar * 37) % N))
    dst_p = jnp.concatenate((edge_index[1], N + (pad_ar % 128)))
    dst2d = dst_p.reshape(E_PAD // CHUNK, CHUNK)
    ids3d = pyg_batch.reshape(GRID, 1, BLK)

    r = lambda v: v.reshape(1, -1)

    h = x
    pools = []
    for (W1, b1, g, be, W2, b2) in (
        (c1_W1, c1_b1, c1_g, c1_be, c1_W2, c1_b2),
        (c2_W1, c2_b1, c2_g, c2_be, c2_W2, c2_b2),
        (c3_W1, c3_b1, c3_g, c3_be, c3_W2, c3_b2),
    ):
        part = _segsum(h, src_p, dst2d)
        h, p = _mlp_pool(h, part[0], part[1], ids3d,
                         W1, r(b1), r(g), r(be), W2, r(b2))
        pools.append(p)

    out = _head(pools[0], pools[1], pools[2],
                lin1_W, r(lin1_b), lin2_W.reshape(1, -1), r(lin2_b))
    return out.reshape(B)
